# baseline (speedup 1.0000x reference)
import jax
import jax.numpy as jnp
from jax import lax
from jax.experimental import pallas as pl
from jax.experimental.pallas import tpu as pltpu
from jax.experimental.pallas import tpu_sc as plsc

N_EMB = 128

_NC = 2
_NS = 16
_NW = _NC * _NS

_CH = 128
_NBUF = 5


def _gather_body(idx_hbm, table_hbm, out_hbm, idx_v, rows_v, gsems, wsems):
    per_w = idx_v.shape[0]
    n_chunks = per_w // _CH
    wid = lax.axis_index("s") * _NC + lax.axis_index("c")
    base = wid * per_w
    pltpu.sync_copy(idx_hbm.at[pl.ds(base, per_w)], idx_v)

    def gather(c, b):
        return pltpu.make_async_copy(
            table_hbm.at[idx_v.at[pl.ds(c * _CH, _CH)]],
            rows_v.at[b], gsems[b])

    def write(c, b):
        return pltpu.make_async_copy(
            rows_v.at[b], out_hbm.at[pl.ds(base + c * _CH, _CH)], wsems[b])

    for b in range(_NBUF):
        gather(b, b).start()

    @pl.loop(0, n_chunks // _NBUF)
    def _block(jj):
        j = jj * _NBUF
        for b in range(_NBUF):
            c = j + b
            gather(c, b).wait()
            write(c, b).start()

            @pl.when(c + _NBUF < n_chunks)
            def _():
                write(c, b).wait()
                gather(c + _NBUF, b).start()

    for b in range(_NBUF):
        write(n_chunks - _NBUF + b, b).wait()


def kernel(input, emb_table):
    B, L = input.shape
    total = B * L
    assert total % (_NW * _CH) == 0
    per_w = total // _NW
    assert (per_w // _CH) % _NBUF == 0 and per_w % 8 == 0

    mesh = plsc.VectorSubcoreMesh(core_axis_name="c", subcore_axis_name="s")
    out = pl.kernel(
        _gather_body,
        out_type=jax.ShapeDtypeStruct((total, N_EMB), jnp.float32),
        mesh=mesh,
        scratch_types=[
            pltpu.VMEM((per_w,), jnp.int32),
            pltpu.VMEM((_NBUF, _CH, N_EMB), jnp.float32),
            [pltpu.SemaphoreType.DMA] * _NBUF,
            [pltpu.SemaphoreType.DMA] * _NBUF,
        ],
    )(input.reshape(total), emb_table)
    return out.reshape(B, L, N_EMB)


# speedup vs baseline: 1.0073x; 1.0073x over previous
import jax
import jax.numpy as jnp
from jax import lax
from jax.experimental import pallas as pl
from jax.experimental.pallas import tpu as pltpu
from jax.experimental.pallas import tpu_sc as plsc

N_EMB = 128

_NC = 2
_NS = 16
_NW = _NC * _NS

_NBUF = 4

_SPLITS = ((0, 128), (128, 72))


def _gather_body(idx_hbm, table_hbm, out_hbm, idx_v, rows_v, gsems, wsems):
    rows_per_w, L = idx_v.shape
    per_w = rows_per_w * L
    wid = lax.axis_index("s") * _NC + lax.axis_index("c")
    base = wid * per_w
    pltpu.sync_copy(idx_hbm.at[pl.ds(wid * rows_per_w, rows_per_w)], idx_v)

    def gather(r, s, b):
        off, ln = _SPLITS[s]
        return pltpu.make_async_copy(
            table_hbm.at[idx_v.at[r, pl.ds(off, ln)]],
            rows_v.at[b, pl.ds(0, ln)],
            gsems[b])

    def write(r, s, b):
        off, ln = _SPLITS[s]
        return pltpu.make_async_copy(
            rows_v.at[b, pl.ds(0, ln)],
            out_hbm.at[pl.ds(base + r * L + off, ln)],
            wsems[b])

    n_chunks = rows_per_w * 2

    for b in range(_NBUF):
        gather(b // 2, b % 2, b).start()

    @pl.loop(0, n_chunks // _NBUF)
    def _block(jj):
        j = jj * _NBUF
        for b in range(_NBUF):
            c = j + b
            r, s = c // 2, b % 2
            gather(r, s, b).wait()
            write(r, s, b).start()

            @pl.when(c + _NBUF < n_chunks)
            def _():
                rn = (c + _NBUF) // 2
                write(r, s, b).wait()
                gather(rn, b % 2, b).start()

    for b in range(_NBUF):
        c = n_chunks - _NBUF + b
        write(c // 2, b % 2, b).wait()


def kernel(input, emb_table):
    B, L = input.shape
    total = B * L
    assert B % _NW == 0 and L == 200

    mesh = plsc.VectorSubcoreMesh(core_axis_name="c", subcore_axis_name="s")
    out = pl.kernel(
        _gather_body,
        out_type=jax.ShapeDtypeStruct((total, N_EMB), jnp.float32),
        mesh=mesh,
        compiler_params=pltpu.CompilerParams(use_tc_tiling_on_sc=True),
        scratch_types=[
            pltpu.VMEM((B // _NW, L), jnp.int32),
            pltpu.VMEM((_NBUF, 128, N_EMB), jnp.float32),
            [pltpu.SemaphoreType.DMA] * _NBUF,
            [pltpu.SemaphoreType.DMA] * _NBUF,
        ],
    )(input, emb_table)
    return out.reshape(B, L, N_EMB)
